# baseline (speedup 1.0000x reference)
import functools

import jax
import jax.numpy as jnp
from jax import lax
from jax.experimental import pallas as pl
from jax.experimental.pallas import tpu as pltpu
from jax.experimental.pallas import tpu_sc as plsc

_EPS_BN = 1e-5
_EPS_L2 = 1e-12

_NC = 2
_NS = 16
_NW = _NC * _NS
_CHA = 128
_CHD = 128
_DBUF = 4
_ZR = 8


def _bn_msg_body(x_ref, gamma_ref, beta_ref, q_ref, qb_ref, h_ref, msg_ref):
    x = x_ref[...]
    mu = jnp.mean(x, axis=0, keepdims=True)
    xc = x - mu
    var = jnp.mean(xc * xc, axis=0, keepdims=True)
    h = gamma_ref[...] * xc * lax.rsqrt(var + _EPS_BN) + beta_ref[...]
    h_ref[...] = h
    msg_ref[...] = jnp.maximum(
        jnp.dot(h, q_ref[...], preferred_element_type=jnp.float32) + qb_ref[...], 0.0)


def _combine_msg_body(h_ref, agg_ref, deg_ref, w_ref, wb_ref, q_ref, qb_ref,
                      h2_ref, msg_ref, *, n, d):
    agg = agg_ref[0, :n, :] + agg_ref[1, :n, :]
    deg = deg_ref[0, :n, 0:1] + deg_ref[1, :n, 0:1]
    neigh = agg / jnp.maximum(deg, 1.0)
    h = h_ref[...]
    pre = (jnp.dot(h, w_ref[:d, :], preferred_element_type=jnp.float32)
           + jnp.dot(neigh, w_ref[d:, :], preferred_element_type=jnp.float32)
           + wb_ref[...])
    h2 = jnp.maximum(pre, 0.0)
    nrm = jnp.sqrt(jnp.sum(h2 * h2, axis=1, keepdims=True))
    h2n = h2 / (nrm + _EPS_L2)
    h2_ref[...] = h2n
    msg_ref[...] = jnp.maximum(
        jnp.dot(h2n, q_ref[...], preferred_element_type=jnp.float32) + qb_ref[...], 0.0)


def _final_body(h_ref, agg_ref, deg_ref, w_ref, wb_ref, gw_ref, gb_ref, g_ref,
                gamma_ref, beta_ref, out_ref, *, n, d):
    agg = agg_ref[0, :n, :] + agg_ref[1, :n, :]
    deg = deg_ref[0, :n, 0:1] + deg_ref[1, :n, 0:1]
    neigh = agg / jnp.maximum(deg, 1.0)
    h = h_ref[...]
    pre = (jnp.dot(h, w_ref[:d, :], preferred_element_type=jnp.float32)
           + jnp.dot(neigh, w_ref[d:, :], preferred_element_type=jnp.float32)
           + wb_ref[...])
    h2 = jnp.maximum(pre, 0.0)
    nrm = jnp.sqrt(jnp.sum(h2 * h2, axis=1, keepdims=True))
    h2n = h2 / (nrm + _EPS_L2)
    z = g_ref[0, 0] * jnp.maximum(
        jnp.dot(h2n, gw_ref[...], preferred_element_type=jnp.float32) + gb_ref[...], 0.0)
    mu = jnp.mean(z, axis=0, keepdims=True)
    zc = z - mu
    var = jnp.mean(zc * zc, axis=0, keepdims=True)
    out_ref[...] = gamma_ref[...] * zc * lax.rsqrt(var + _EPS_BN) + beta_ref[...]


def _zero_fill(zbuf, width):
    for i in range(_ZR):
        for j in range(width // 16):
            zbuf[i, pl.ds(j * 16, 16)] = jnp.zeros((16,), jnp.float32)


def _make_edge_agg(n_pad, epw, msg_d):
    rpt = n_pad // _NS
    n_chunks = epw // _CHA
    mesh = plsc.VectorSubcoreMesh(core_axis_name="c", subcore_axis_name="s")

    @functools.partial(
        pl.kernel, mesh=mesh,
        out_type=jax.ShapeDtypeStruct((_NC * n_pad, msg_d), jnp.float32),
        scratch_types=[
            pltpu.VMEM((_CHA,), jnp.int32),
            pltpu.VMEM((_CHA,), jnp.int32),
            pltpu.VMEM((_ZR, msg_d), jnp.float32),
            pltpu.VMEM_SHARED((n_pad, msg_d), jnp.float32),
            pltpu.VMEM((_CHA, msg_d), jnp.float32),
            pltpu.SemaphoreType.DMA,
        ])
    def edge_agg(msg_hbm, src_hbm, dst_hbm, agg_out,
                 src_v, dst_v, zbuf_v, agg_sh, rows_v, sem):
        c = lax.axis_index("c")
        s = lax.axis_index("s")
        wid = s * _NC + c
        _zero_fill(zbuf_v, msg_d)

        def zbody(k, carry):
            pltpu.sync_copy(zbuf_v, agg_sh.at[pl.ds(s * rpt + k * _ZR, _ZR)])
            return carry

        lax.fori_loop(0, rpt // _ZR, zbody, 0)
        plsc.subcore_barrier()

        def body(i, carry):
            base = wid * epw + i * _CHA
            pltpu.sync_copy(src_hbm.at[pl.ds(base, _CHA)], src_v)
            pltpu.sync_copy(dst_hbm.at[pl.ds(base, _CHA)], dst_v)
            pltpu.async_copy(msg_hbm.at[src_v], rows_v, sem).wait()
            pltpu.sync_copy(rows_v, agg_sh.at[dst_v], add=True)
            return carry

        lax.fori_loop(0, n_chunks, body, 0)
        plsc.subcore_barrier()
        pltpu.sync_copy(agg_sh.at[pl.ds(s * rpt, rpt)],
                        agg_out.at[pl.ds(c * n_pad + s * rpt, rpt)])

    return edge_agg


def _make_deg_hist(n_pad, epw, msg_d):
    rpt = n_pad // _NS
    n_chunks = epw // _CHD
    mesh = plsc.VectorSubcoreMesh(core_axis_name="c", subcore_axis_name="s")

    @functools.partial(
        pl.kernel, mesh=mesh,
        out_type=jax.ShapeDtypeStruct((_NC * n_pad, msg_d), jnp.float32),
        scratch_types=[
            pltpu.VMEM((_CHD,), jnp.int32),
            pltpu.VMEM((_CHD, msg_d), jnp.float32),
            pltpu.VMEM((_ZR, msg_d), jnp.float32),
            pltpu.VMEM_SHARED((n_pad, msg_d), jnp.float32),
        ])
    def deg_hist(dst_hbm, ones_hbm, deg_out, dst_v, ones_v, zbuf_v, deg_sh):
        c = lax.axis_index("c")
        s = lax.axis_index("s")
        wid = s * _NC + c
        _zero_fill(zbuf_v, msg_d)
        pltpu.sync_copy(ones_hbm, ones_v)

        def zbody(k, carry):
            pltpu.sync_copy(zbuf_v, deg_sh.at[pl.ds(s * rpt + k * _ZR, _ZR)])
            return carry

        lax.fori_loop(0, rpt // _ZR, zbody, 0)
        plsc.subcore_barrier()

        def body(i, carry):
            base = wid * epw + i * _CHD
            pltpu.sync_copy(dst_hbm.at[pl.ds(base, _CHD)], dst_v)
            pltpu.sync_copy(ones_v, deg_sh.at[dst_v], add=True)
            return carry

        lax.fori_loop(0, n_chunks, body, 0)
        plsc.subcore_barrier()
        pltpu.sync_copy(deg_sh.at[pl.ds(s * rpt, rpt)],
                        deg_out.at[pl.ds(c * n_pad + s * rpt, rpt)])

    return deg_hist


def kernel(x, Q1, qb1, W1, wb1, Q2, qb2, W2, wb2, GW, Gb, g,
           gamma_in, beta_in, gamma_out, beta_out, edge_index):
    n, d = x.shape
    h_dim = Q1.shape[1]
    o_dim = W1.shape[1]
    out_dim = GW.shape[1]
    e = edge_index.shape[1]

    n_pad = ((n + 16 + (_NS * _ZR) - 1) // (_NS * _ZR)) * (_NS * _ZR)
    quant = 16 * _CHA
    epw = ((e + _NW * quant - 1) // (_NW * quant)) * quant
    e_pad = _NW * epw

    src = edge_index[0]
    dst = edge_index[1]
    pad_e = e_pad - e
    src_p = jnp.concatenate([src, jnp.zeros((pad_e,), jnp.int32)])
    dst_p = jnp.concatenate([dst, jnp.full((pad_e,), n, jnp.int32)])
    ones_blk = jnp.ones((_CHD, h_dim), jnp.float32)

    bn_msg = pl.pallas_call(
        _bn_msg_body,
        out_shape=[jax.ShapeDtypeStruct((n, d), jnp.float32),
                   jax.ShapeDtypeStruct((n, h_dim), jnp.float32)],
    )
    h0, msg1 = bn_msg(x, gamma_in.reshape(1, d), beta_in.reshape(1, d),
                      Q1, qb1.reshape(1, h_dim))

    deg_hist = _make_deg_hist(n_pad, epw, h_dim)
    deg = deg_hist(dst_p, ones_blk).reshape(_NC, n_pad, h_dim)

    edge_agg = _make_edge_agg(n_pad, epw, h_dim)
    agg1 = edge_agg(msg1, src_p, dst_p).reshape(_NC, n_pad, h_dim)

    combine1 = pl.pallas_call(
        functools.partial(_combine_msg_body, n=n, d=d),
        out_shape=[jax.ShapeDtypeStruct((n, o_dim), jnp.float32),
                   jax.ShapeDtypeStruct((n, Q2.shape[1]), jnp.float32)],
    )
    h1, msg2 = combine1(h0, agg1, deg, W1, wb1.reshape(1, o_dim),
                        Q2, qb2.reshape(1, Q2.shape[1]))

    agg2 = edge_agg(msg2, src_p, dst_p).reshape(_NC, n_pad, Q2.shape[1])

    final = pl.pallas_call(
        functools.partial(_final_body, n=n, d=o_dim),
        out_shape=jax.ShapeDtypeStruct((n, out_dim), jnp.float32),
    )
    out = final(h1, agg2, deg, W2, wb2.reshape(1, o_dim), GW,
                Gb.reshape(1, out_dim), g.reshape(1, 1), gamma_out.reshape(1, out_dim),
                beta_out.reshape(1, out_dim))
    return out


# speedup vs baseline: 1.4325x; 1.4325x over previous
import functools

import jax
import jax.numpy as jnp
from jax import lax
from jax.experimental import pallas as pl
from jax.experimental.pallas import tpu as pltpu
from jax.experimental.pallas import tpu_sc as plsc

_EPS_BN = 1e-5
_EPS_L2 = 1e-12

_NC = 2
_NS = 16
_NW = _NC * _NS
_CHA = 128
_CHD = 128
_DBUF = 4
_ZR = 8


def _bn_msg_body(x_ref, gamma_ref, beta_ref, q_ref, qb_ref, h_ref, msg_ref):
    x = x_ref[...]
    mu = jnp.mean(x, axis=0, keepdims=True)
    xc = x - mu
    var = jnp.mean(xc * xc, axis=0, keepdims=True)
    h = gamma_ref[...] * xc * lax.rsqrt(var + _EPS_BN) + beta_ref[...]
    h_ref[...] = h
    msg_ref[...] = jnp.maximum(
        jnp.dot(h, q_ref[...], preferred_element_type=jnp.float32) + qb_ref[...], 0.0)


def _combine_msg_body(h_ref, agg_ref, deg_ref, w_ref, wb_ref, q_ref, qb_ref,
                      h2_ref, msg_ref, *, n, d):
    agg = agg_ref[0, :n, :] + agg_ref[1, :n, :]
    deg = deg_ref[0, :n, 0:1] + deg_ref[1, :n, 0:1]
    neigh = agg / jnp.maximum(deg, 1.0)
    h = h_ref[...]
    pre = (jnp.dot(h, w_ref[:d, :], preferred_element_type=jnp.float32)
           + jnp.dot(neigh, w_ref[d:, :], preferred_element_type=jnp.float32)
           + wb_ref[...])
    h2 = jnp.maximum(pre, 0.0)
    nrm = jnp.sqrt(jnp.sum(h2 * h2, axis=1, keepdims=True))
    h2n = h2 / (nrm + _EPS_L2)
    h2_ref[...] = h2n
    msg_ref[...] = jnp.maximum(
        jnp.dot(h2n, q_ref[...], preferred_element_type=jnp.float32) + qb_ref[...], 0.0)


def _final_body(h_ref, agg_ref, deg_ref, w_ref, wb_ref, gw_ref, gb_ref, g_ref,
                gamma_ref, beta_ref, out_ref, *, n, d):
    agg = agg_ref[0, :n, :] + agg_ref[1, :n, :]
    deg = deg_ref[0, :n, 0:1] + deg_ref[1, :n, 0:1]
    neigh = agg / jnp.maximum(deg, 1.0)
    h = h_ref[...]
    pre = (jnp.dot(h, w_ref[:d, :], preferred_element_type=jnp.float32)
           + jnp.dot(neigh, w_ref[d:, :], preferred_element_type=jnp.float32)
           + wb_ref[...])
    h2 = jnp.maximum(pre, 0.0)
    nrm = jnp.sqrt(jnp.sum(h2 * h2, axis=1, keepdims=True))
    h2n = h2 / (nrm + _EPS_L2)
    z = g_ref[0, 0] * jnp.maximum(
        jnp.dot(h2n, gw_ref[...], preferred_element_type=jnp.float32) + gb_ref[...], 0.0)
    mu = jnp.mean(z, axis=0, keepdims=True)
    zc = z - mu
    var = jnp.mean(zc * zc, axis=0, keepdims=True)
    out_ref[...] = gamma_ref[...] * zc * lax.rsqrt(var + _EPS_BN) + beta_ref[...]


def _zero_fill(zbuf, width):
    for i in range(_ZR):
        for j in range(width // 16):
            zbuf[i, pl.ds(j * 16, 16)] = jnp.zeros((16,), jnp.float32)


def _make_edge_agg(n_pad, epw, msg_d):
    rpt = n_pad // _NS
    n_chunks = epw // _CHA
    mesh = plsc.VectorSubcoreMesh(core_axis_name="c", subcore_axis_name="s")

    @functools.partial(
        pl.kernel, mesh=mesh,
        out_type=jax.ShapeDtypeStruct((_NC * n_pad, msg_d), jnp.float32),
        scratch_types=[
            pltpu.VMEM((_CHA,), jnp.int32),
            pltpu.VMEM((_CHA,), jnp.int32),
            pltpu.VMEM((_CHA, msg_d), jnp.float32),
            pltpu.VMEM((_ZR, msg_d), jnp.float32),
            pltpu.VMEM_SHARED((n_pad, msg_d), jnp.float32),
            pltpu.SemaphoreType.DMA,
        ])
    def edge_agg(msg_hbm, src_hbm, dst_hbm, agg_out,
                 src_v, dst_v, rows_v, zbuf_v, agg_sh, sem):
        c = lax.axis_index("c")
        s = lax.axis_index("s")
        wid = s * _NC + c
        _zero_fill(zbuf_v, msg_d)

        def zbody(k, carry):
            pltpu.sync_copy(zbuf_v, agg_sh.at[pl.ds(s * rpt + k * _ZR, _ZR)])
            return carry

        lax.fori_loop(0, rpt // _ZR, zbody, 0)
        plsc.subcore_barrier()

        def body(i, carry):
            base = wid * epw + i * _CHA
            pltpu.sync_copy(src_hbm.at[pl.ds(base, _CHA)], src_v)
            pltpu.sync_copy(dst_hbm.at[pl.ds(base, _CHA)], dst_v)
            pltpu.async_copy(msg_hbm.at[src_v], rows_v, sem).wait()
            pltpu.sync_copy(rows_v, agg_sh.at[dst_v], add=True)
            return carry

        lax.fori_loop(0, n_chunks, body, 0)
        plsc.subcore_barrier()
        pltpu.sync_copy(agg_sh.at[pl.ds(s * rpt, rpt)],
                        agg_out.at[pl.ds(c * n_pad + s * rpt, rpt)])

    return edge_agg


def _make_deg_hist(n_pad, epw, msg_d):
    rpt = n_pad // _NS
    n_chunks = epw // _CHD
    mesh = plsc.VectorSubcoreMesh(core_axis_name="c", subcore_axis_name="s")

    @functools.partial(
        pl.kernel, mesh=mesh,
        out_type=jax.ShapeDtypeStruct((_NC * n_pad, msg_d), jnp.float32),
        scratch_types=[
            pltpu.VMEM((_CHD,), jnp.int32),
            pltpu.VMEM((_CHD, msg_d), jnp.float32),
            pltpu.VMEM((_ZR, msg_d), jnp.float32),
            pltpu.VMEM_SHARED((n_pad, msg_d), jnp.float32),
        ])
    def deg_hist(dst_hbm, ones_hbm, deg_out, dst_v, ones_v, zbuf_v, deg_sh):
        c = lax.axis_index("c")
        s = lax.axis_index("s")
        wid = s * _NC + c
        _zero_fill(zbuf_v, msg_d)
        pltpu.sync_copy(ones_hbm, ones_v)

        def zbody(k, carry):
            pltpu.sync_copy(zbuf_v, deg_sh.at[pl.ds(s * rpt + k * _ZR, _ZR)])
            return carry

        lax.fori_loop(0, rpt // _ZR, zbody, 0)
        plsc.subcore_barrier()

        def body(i, carry):
            base = wid * epw + i * _CHD
            pltpu.sync_copy(dst_hbm.at[pl.ds(base, _CHD)], dst_v)
            pltpu.sync_copy(ones_v, deg_sh.at[dst_v], add=True)
            return carry

        lax.fori_loop(0, n_chunks, body, 0)
        plsc.subcore_barrier()
        pltpu.sync_copy(deg_sh.at[pl.ds(s * rpt, rpt)],
                        deg_out.at[pl.ds(c * n_pad + s * rpt, rpt)])

    return deg_hist


def kernel(x, Q1, qb1, W1, wb1, Q2, qb2, W2, wb2, GW, Gb, g,
           gamma_in, beta_in, gamma_out, beta_out, edge_index):
    n, d = x.shape
    h_dim = Q1.shape[1]
    o_dim = W1.shape[1]
    out_dim = GW.shape[1]
    e = edge_index.shape[1]

    n_pad = ((n + 16 + (_NS * _ZR) - 1) // (_NS * _ZR)) * (_NS * _ZR)
    epw = ((e + _NW * _CHA - 1) // (_NW * _CHA)) * _CHA
    e_pad = _NW * epw

    src = edge_index[0]
    dst = edge_index[1]
    pad_e = e_pad - e
    src_p = jnp.concatenate([src, jnp.zeros((pad_e,), jnp.int32)])
    dst_p = jnp.concatenate([dst, jnp.full((pad_e,), n, jnp.int32)])
    ones_blk = jnp.ones((_CHD, h_dim), jnp.float32)

    bn_msg = pl.pallas_call(
        _bn_msg_body,
        out_shape=[jax.ShapeDtypeStruct((n, d), jnp.float32),
                   jax.ShapeDtypeStruct((n, h_dim), jnp.float32)],
    )
    h0, msg1 = bn_msg(x, gamma_in.reshape(1, d), beta_in.reshape(1, d),
                      Q1, qb1.reshape(1, h_dim))

    deg_hist = _make_deg_hist(n_pad, epw, h_dim)
    deg = deg_hist(dst_p, ones_blk).reshape(_NC, n_pad, h_dim)

    edge_agg = _make_edge_agg(n_pad, epw, h_dim)
    agg1 = edge_agg(msg1, src_p, dst_p).reshape(_NC, n_pad, h_dim)

    combine1 = pl.pallas_call(
        functools.partial(_combine_msg_body, n=n, d=d),
        out_shape=[jax.ShapeDtypeStruct((n, o_dim), jnp.float32),
                   jax.ShapeDtypeStruct((n, Q2.shape[1]), jnp.float32)],
    )
    h1, msg2 = combine1(h0, agg1, deg, W1, wb1.reshape(1, o_dim),
                        Q2, qb2.reshape(1, Q2.shape[1]))

    agg2 = edge_agg(msg2, src_p, dst_p).reshape(_NC, n_pad, Q2.shape[1])

    final = pl.pallas_call(
        functools.partial(_final_body, n=n, d=o_dim),
        out_shape=jax.ShapeDtypeStruct((n, out_dim), jnp.float32),
    )
    out = final(h1, agg2, deg, W2, wb2.reshape(1, o_dim), GW,
                Gb.reshape(1, out_dim), g.reshape(1, 1), gamma_out.reshape(1, out_dim),
                beta_out.reshape(1, out_dim))
    return out
